# baseline (speedup 1.0000x reference)
import jax
import jax.numpy as jnp
from jax.experimental import pallas as pl
from jax.experimental.pallas import tpu as pltpu

_TILE = 512
_NTOK = 8192
_NBLK = _NTOK // _TILE
_NBUF = 4


def _gating_kernel(x_hbm, w_ref, b_ref, o_hbm):
    def inner(x_blk, o_blk):
        logits = jax.lax.dot_general(
            x_blk[...], w_ref[...],
            dimension_numbers=(((1,), (1,)), ((), ())),
            preferred_element_type=jnp.float32,
        )
        logits = logits + b_ref[...]
        m = jnp.max(logits, axis=-1, keepdims=True)
        e = jnp.exp(logits - m)
        s = jnp.sum(e, axis=-1, keepdims=True)
        o_blk[...] = e / s

    pipe = pltpu.emit_pipeline(
        inner,
        grid=(_NBLK,),
        in_specs=[
            pl.BlockSpec((_TILE, 2048), lambda i: (i, 0),
                         pipeline_mode=pl.Buffered(
                             buffer_count=_NBUF, use_lookahead=True)),
        ],
        out_specs=[
            pl.BlockSpec((_TILE, 64), lambda i: (i, 0)),
        ],
    )
    pipe(x_hbm, o_hbm)


def kernel(x, W, b):
    n_tokens, input_dim = x.shape
    num_experts = W.shape[0]
    b2 = b.reshape(1, num_experts)
    return pl.pallas_call(
        _gating_kernel,
        in_specs=[
            pl.BlockSpec(memory_space=pltpu.MemorySpace.HBM),
            pl.BlockSpec(memory_space=pltpu.MemorySpace.VMEM),
            pl.BlockSpec(memory_space=pltpu.MemorySpace.VMEM),
        ],
        out_specs=pl.BlockSpec(memory_space=pltpu.MemorySpace.HBM),
        out_shape=jax.ShapeDtypeStruct((n_tokens, num_experts), jnp.float32),
    )(x, W, b2)
